# baseline (speedup 1.0000x reference)
import functools

import jax
import jax.numpy as jnp
from jax import lax
from jax.experimental import pallas as pl
from jax.experimental.pallas import tpu as pltpu
from jax.experimental.pallas import tpu_sc as plsc

E = 8
TOP_K = 2
D_MODEL = 1024
MOE_FF = 512
SHARED_FF = 1024
T = 2048

TILE = 256
CAP = 2048
BLOCKS_PER_E = CAP // TILE
MAX_TILES = 24
DUMP_BLOCK = E * BLOCKS_PER_E
N_BUF = (DUMP_BLOCK + 1) * TILE

NW = 32
TOK_PER_W = T // NW


def _silu(x):
    return x * jax.nn.sigmoid(x)


def _router_body(x_ref, gate_wt_ref, slots_ref, w0e_ref, w1e_ref, meta_ref,
                 run_ref):
    i = pl.program_id(0)
    x = x_ref[...]
    logits = lax.dot_general(
        x, gate_wt_ref[...], (((1,), (0,)), ((), ())),
        preferred_element_type=jnp.float32)
    m = jnp.max(logits, axis=-1, keepdims=True)
    ex = jnp.exp(logits - m)
    probs = ex / jnp.sum(ex, axis=-1, keepdims=True)

    iota = lax.broadcasted_iota(jnp.int32, probs.shape, 1)
    p1 = jnp.max(probs, axis=-1, keepdims=True)
    id1 = jnp.min(jnp.where(probs == p1, iota, E), axis=-1, keepdims=True)
    oh1 = iota == id1
    probs_m = jnp.where(oh1, -jnp.inf, probs)
    p2 = jnp.max(probs_m, axis=-1, keepdims=True)
    id2 = jnp.min(jnp.where(probs_m == p2, iota, E), axis=-1, keepdims=True)
    oh2 = iota == id2
    denom = p1 + p2
    w0e_ref[...] = jnp.broadcast_to(p1 / denom, (TILE, 16))
    w1e_ref[...] = jnp.broadcast_to(p2 / denom, (TILE, 16))

    a = (oh1 | oh2).astype(jnp.float32)

    ri = lax.broadcasted_iota(jnp.int32, (TILE, TILE), 0)
    ci = lax.broadcasted_iota(jnp.int32, (TILE, TILE), 1)
    tri = (ci < ri).astype(jnp.bfloat16)
    excl = lax.dot_general(
        tri, a.astype(jnp.bfloat16), (((1,), (0,)), ((), ())),
        preferred_element_type=jnp.float32)

    @pl.when(i == 0)
    def _():
        run_ref[...] = jnp.zeros_like(run_ref)

    run = run_ref[...]
    pos = excl + run
    new_run = run + jnp.sum(a, axis=0, keepdims=True)
    run_ref[...] = new_run

    eoff = lax.broadcasted_iota(jnp.int32, (TILE, E), 1).astype(jnp.float32) * CAP
    slotmat = pos + eoff
    slot0 = jnp.sum(jnp.where(oh1, slotmat, 0.0), axis=1)
    slot1 = jnp.sum(jnp.where(oh2, slotmat, 0.0), axis=1)

    pad6 = jnp.zeros((6, TILE), jnp.float32)
    slots_ref[...] = jnp.concatenate(
        [slot0[None, :], slot1[None, :], pad6], axis=0).astype(jnp.int32)

    @pl.when(i == pl.num_programs(0) - 1)
    def _():
        counts = new_run
        ntiles = jnp.floor((counts + (TILE - 1)) / TILE)
        e1 = lax.broadcasted_iota(jnp.int32, (E, E), 0)
        e2 = lax.broadcasted_iota(jnp.int32, (E, E), 1)
        strict = (e1 < e2).astype(jnp.float32)
        cum_excl = lax.dot_general(
            ntiles, strict, (((1,), (0,)), ((), ())),
            preferred_element_type=jnp.float32)
        nvalid = jnp.sum(ntiles, axis=-1, keepdims=True)

        ti = lax.broadcasted_iota(jnp.int32, (MAX_TILES_PAD, E), 0).astype(jnp.float32)
        ge = ti >= cum_excl
        texp = jnp.sum(ge.astype(jnp.float32), axis=1) - 1.0
        base = jnp.max(jnp.where(ge, jnp.broadcast_to(cum_excl, ge.shape),
                                 -1.0), axis=1)
        i32col = lax.broadcasted_iota(jnp.int32, (MAX_TILES_PAD,), 0).astype(jnp.float32)
        tblock = texp * BLOCKS_PER_E + (i32col - base)
        invalid = i32col >= nvalid[0, 0]
        tblock = jnp.where(invalid, float(DUMP_BLOCK), tblock)
        texp = jnp.where(invalid, 0.0, texp)
        pad = jnp.zeros((META_W - 2 * MAX_TILES_PAD,), jnp.float32)
        meta_ref[...] = jnp.concatenate(
            [texp, tblock, pad], axis=0)[None, :].astype(jnp.int32)


MAX_TILES_PAD = 32
META_W = 128


def _router(hidden, gate_wt):
    grid = T // TILE
    return pl.pallas_call(
        _router_body,
        grid=(grid,),
        in_specs=[
            pl.BlockSpec((TILE, D_MODEL), lambda i: (i, 0)),
            pl.BlockSpec((D_MODEL, E), lambda i: (0, 0)),
        ],
        out_specs=[
            pl.BlockSpec((8, TILE), lambda i: (0, i)),
            pl.BlockSpec((TILE, 16), lambda i: (i, 0)),
            pl.BlockSpec((TILE, 16), lambda i: (i, 0)),
            pl.BlockSpec((1, META_W), lambda i: (0, 0)),
        ],
        out_shape=[
            jax.ShapeDtypeStruct((8, T), jnp.int32),
            jax.ShapeDtypeStruct((T, 16), jnp.float32),
            jax.ShapeDtypeStruct((T, 16), jnp.float32),
            jax.ShapeDtypeStruct((1, META_W), jnp.int32),
        ],
        scratch_shapes=[pltpu.VMEM((1, E), jnp.float32)],
    )(hidden, gate_wt)


def _shared_body(x_ref, sgu_ref, sdown_ref, out_ref):
    xb = x_ref[...].astype(jnp.bfloat16)
    h = jnp.dot(xb, sgu_ref[...], preferred_element_type=jnp.float32)
    g = h[:, :SHARED_FF]
    u = h[:, SHARED_FF:]
    act = (_silu(g) * u).astype(jnp.bfloat16)
    out_ref[...] = jnp.dot(act, sdown_ref[...],
                           preferred_element_type=jnp.float32)


def _shared(hidden, sgu, sdown):
    grid = T // TILE
    return pl.pallas_call(
        _shared_body,
        grid=(grid,),
        in_specs=[
            pl.BlockSpec((TILE, D_MODEL), lambda i: (i, 0)),
            pl.BlockSpec((D_MODEL, 2 * SHARED_FF), lambda i: (0, 0)),
            pl.BlockSpec((SHARED_FF, D_MODEL), lambda i: (0, 0)),
        ],
        out_specs=pl.BlockSpec((TILE, D_MODEL), lambda i: (i, 0)),
        out_shape=jax.ShapeDtypeStruct((T, D_MODEL), jnp.float32),
    )(hidden, sgu, sdown)


def _dispatch_body(hid_hbm, s0_hbm, s1_hbm, xs_hbm, hid_v, s0_v, s1_v, sem):
    wid = lax.axis_index("s") * 2 + lax.axis_index("c")
    base = wid * TOK_PER_W
    pltpu.sync_copy(hid_hbm.at[pl.ds(base, TOK_PER_W)], hid_v)
    pltpu.sync_copy(s0_hbm.at[pl.ds(base, TOK_PER_W)], s0_v)
    pltpu.sync_copy(s1_hbm.at[pl.ds(base, TOK_PER_W)], s1_v)
    c0 = pltpu.make_async_copy(hid_v, xs_hbm.at[s0_v], sem)
    c0.start()
    c1 = pltpu.make_async_copy(hid_v, xs_hbm.at[s1_v], sem)
    c1.start()
    c0.wait()
    c1.wait()


def _dispatch(hidden, s0, s1):
    mesh = plsc.VectorSubcoreMesh(core_axis_name="c", subcore_axis_name="s", num_cores=2, num_subcores=16)
    f = pl.kernel(
        _dispatch_body,
        out_type=jax.ShapeDtypeStruct((N_BUF, D_MODEL), jnp.float32),
        mesh=mesh,
        scratch_types=[
            pltpu.VMEM((TOK_PER_W, D_MODEL), jnp.float32),
            pltpu.VMEM((TOK_PER_W,), jnp.int32),
            pltpu.VMEM((TOK_PER_W,), jnp.int32),
            pltpu.SemaphoreType.DMA,
        ],
    )
    return f(hidden, s0, s1)


def _gmm_body(texp_ref, tblock_ref, x_ref, wgu_ref, wdown_ref, y_ref):
    i = pl.program_id(0)

    @pl.when(tblock_ref[i] != DUMP_BLOCK)
    def _():
        xb = x_ref[...].astype(jnp.bfloat16)
        h = jnp.dot(xb, wgu_ref[0], preferred_element_type=jnp.float32)
        g = h[:, :MOE_FF]
        u = h[:, MOE_FF:]
        act = (_silu(g) * u).astype(jnp.bfloat16)
        y_ref[...] = jnp.dot(act, wdown_ref[0],
                             preferred_element_type=jnp.float32)


def _gmm(texp, tblock, x_sorted, wgu, wdown):
    grid_spec = pltpu.PrefetchScalarGridSpec(
        num_scalar_prefetch=2,
        grid=(MAX_TILES,),
        in_specs=[
            pl.BlockSpec((TILE, D_MODEL), lambda i, texp, tblock: (tblock[i], 0)),
            pl.BlockSpec((1, D_MODEL, 2 * MOE_FF),
                         lambda i, texp, tblock: (texp[i], 0, 0)),
            pl.BlockSpec((1, MOE_FF, D_MODEL),
                         lambda i, texp, tblock: (texp[i], 0, 0)),
        ],
        out_specs=pl.BlockSpec((TILE, D_MODEL),
                               lambda i, texp, tblock: (tblock[i], 0)),
    )
    return pl.pallas_call(
        _gmm_body,
        grid_spec=grid_spec,
        out_shape=jax.ShapeDtypeStruct((N_BUF, D_MODEL), jnp.float32),
    )(texp, tblock, x_sorted, wgu, wdown)


def _combine_body(y_hbm, s0_hbm, s1_hbm, w0_hbm, w1_hbm, sh_hbm, out_hbm,
                  g0_v, g1_v, sh_v, o_v, s0_v, s1_v, w0_v, w1_v, sem):
    wid = lax.axis_index("s") * 2 + lax.axis_index("c")
    for chunk in range(TOK_PER_W // 16):
        base = wid * TOK_PER_W + chunk * 16
        pltpu.sync_copy(s0_hbm.at[pl.ds(base, 16)], s0_v)
        pltpu.sync_copy(s1_hbm.at[pl.ds(base, 16)], s1_v)
        pltpu.sync_copy(w0_hbm.at[pl.ds(base, 16)], w0_v)
        pltpu.sync_copy(w1_hbm.at[pl.ds(base, 16)], w1_v)
        pltpu.sync_copy(sh_hbm.at[pl.ds(base, 16)], sh_v)
        c0 = pltpu.make_async_copy(y_hbm.at[s0_v], g0_v, sem)
        c0.start()
        c1 = pltpu.make_async_copy(y_hbm.at[s1_v], g1_v, sem)
        c1.start()
        c0.wait()
        c1.wait()

        def tok_body(j, _):
            w0s = w0_v[j, :]
            w1s = w1_v[j, :]

            def col_body(c, __):
                sl = pl.ds(c * 16, 16)
                o_v[j, sl] = (sh_v[j, sl] + w0s * g0_v[j, sl]
                              + w1s * g1_v[j, sl])
                return 0

            lax.fori_loop(0, D_MODEL // 16, col_body, 0)
            return 0

        lax.fori_loop(0, 16, tok_body, 0)
        pltpu.sync_copy(o_v, out_hbm.at[pl.ds(base, 16)])


def _combine(y, s0, s1, w0, w1, shared_out):
    mesh = plsc.VectorSubcoreMesh(core_axis_name="c", subcore_axis_name="s", num_cores=2, num_subcores=16)
    f = pl.kernel(
        _combine_body,
        out_type=jax.ShapeDtypeStruct((T, D_MODEL), jnp.float32),
        mesh=mesh,
        scratch_types=[
            pltpu.VMEM((16, D_MODEL), jnp.float32),
            pltpu.VMEM((16, D_MODEL), jnp.float32),
            pltpu.VMEM((16, D_MODEL), jnp.float32),
            pltpu.VMEM((16, D_MODEL), jnp.float32),
            pltpu.VMEM((16,), jnp.int32),
            pltpu.VMEM((16,), jnp.int32),
            pltpu.VMEM((16, 16), jnp.float32),
            pltpu.VMEM((16, 16), jnp.float32),
            pltpu.SemaphoreType.DMA,
        ],
    )
    return f(y, s0, s1, w0, w1, shared_out)


@jax.jit
def kernel(hidden_states, gate_w, w_gate_up, w_down, shared_gate_up,
           shared_down):
    gate_wt = gate_w.T
    sgu = shared_gate_up.astype(jnp.bfloat16)
    sdown = shared_down.astype(jnp.bfloat16)
    wgu = w_gate_up.astype(jnp.bfloat16)
    wdown = w_down.astype(jnp.bfloat16)

    slots, w0e, w1e, meta = _router(hidden_states, gate_wt)
    s0 = slots[0]
    s1 = slots[1]
    texp = meta[0, :MAX_TILES_PAD]
    tblock = meta[0, MAX_TILES_PAD:2 * MAX_TILES_PAD]

    shared_out = _shared(hidden_states, sgu, sdown)
    x_sorted = _dispatch(hidden_states, s0, s1)
    y = _gmm(texp, tblock, x_sorted, wgu, wdown)
    return _combine(y, s0, s1, w0e, w1e, shared_out)


# speedup vs baseline: 1.0177x; 1.0177x over previous
import functools

import jax
import jax.numpy as jnp
from jax import lax
from jax.experimental import pallas as pl
from jax.experimental.pallas import tpu as pltpu
from jax.experimental.pallas import tpu_sc as plsc

E = 8
TOP_K = 2
D_MODEL = 1024
MOE_FF = 512
SHARED_FF = 1024
T = 2048

TILE = 256
CAP = 2048
BLOCKS_PER_E = CAP // TILE
MAX_TILES = 24
DUMP_BLOCK = E * BLOCKS_PER_E
N_BUF = (DUMP_BLOCK + 1) * TILE

NW = 32
TOK_PER_W = T // NW


def _silu(x):
    return x * jax.nn.sigmoid(x)


def _router_body(x_ref, gate_wt_ref, slots_ref, w0e_ref, w1e_ref,
                 meta_ref, run_ref):
    i = pl.program_id(0)
    x = x_ref[...]
    logits = lax.dot_general(
        x, gate_wt_ref[...], (((1,), (0,)), ((), ())),
        preferred_element_type=jnp.float32)
    m = jnp.max(logits, axis=-1, keepdims=True)
    ex = jnp.exp(logits - m)
    probs = ex / jnp.sum(ex, axis=-1, keepdims=True)

    iota = lax.broadcasted_iota(jnp.int32, probs.shape, 1)
    p1 = jnp.max(probs, axis=-1, keepdims=True)
    id1 = jnp.min(jnp.where(probs == p1, iota, E), axis=-1, keepdims=True)
    oh1 = iota == id1
    probs_m = jnp.where(oh1, -jnp.inf, probs)
    p2 = jnp.max(probs_m, axis=-1, keepdims=True)
    id2 = jnp.min(jnp.where(probs_m == p2, iota, E), axis=-1, keepdims=True)
    oh2 = iota == id2
    denom = p1 + p2
    w0e_ref[...] = jnp.broadcast_to(p1 / denom, (TILE, 16))
    w1e_ref[...] = jnp.broadcast_to(p2 / denom, (TILE, 16))

    a = (oh1 | oh2).astype(jnp.float32)

    ri = lax.broadcasted_iota(jnp.int32, (TILE, TILE), 0)
    ci = lax.broadcasted_iota(jnp.int32, (TILE, TILE), 1)
    tri = (ci < ri).astype(jnp.bfloat16)
    excl = lax.dot_general(
        tri, a.astype(jnp.bfloat16), (((1,), (0,)), ((), ())),
        preferred_element_type=jnp.float32)

    @pl.when(i == 0)
    def _():
        run_ref[...] = jnp.zeros_like(run_ref)

    run = run_ref[...]
    pos = excl + run
    new_run = run + jnp.sum(a, axis=0, keepdims=True)
    run_ref[...] = new_run

    eoff = lax.broadcasted_iota(jnp.int32, (TILE, E), 1).astype(jnp.float32) * CAP
    slotmat = pos + eoff
    slot0 = jnp.sum(jnp.where(oh1, slotmat, 0.0), axis=1)
    slot1 = jnp.sum(jnp.where(oh2, slotmat, 0.0), axis=1)

    pad6 = jnp.zeros((6, TILE), jnp.float32)
    slots_ref[...] = jnp.concatenate(
        [slot0[None, :], slot1[None, :], pad6], axis=0).astype(jnp.int32)

    @pl.when(i == pl.num_programs(0) - 1)
    def _():
        counts = new_run
        ntiles = jnp.floor((counts + (TILE - 1)) / TILE)
        e1 = lax.broadcasted_iota(jnp.int32, (E, E), 0)
        e2 = lax.broadcasted_iota(jnp.int32, (E, E), 1)
        strict = (e1 < e2).astype(jnp.float32)
        cum_excl = lax.dot_general(
            ntiles, strict, (((1,), (0,)), ((), ())),
            preferred_element_type=jnp.float32)
        nvalid = jnp.sum(ntiles, axis=-1, keepdims=True)

        ti = lax.broadcasted_iota(jnp.int32, (MAX_TILES_PAD, E), 0).astype(jnp.float32)
        ge = ti >= cum_excl
        texp = jnp.sum(ge.astype(jnp.float32), axis=1) - 1.0
        base = jnp.max(jnp.where(ge, jnp.broadcast_to(cum_excl, ge.shape),
                                 -1.0), axis=1)
        i32col = lax.broadcasted_iota(jnp.int32, (MAX_TILES_PAD,), 0).astype(jnp.float32)
        tblock = texp * BLOCKS_PER_E + (i32col - base)
        invalid = i32col >= nvalid[0, 0]
        tblock = jnp.where(invalid, float(DUMP_BLOCK), tblock)
        texp = jnp.where(invalid, 0.0, texp)
        pad = jnp.zeros((META_W - 2 * MAX_TILES_PAD,), jnp.float32)
        meta_ref[...] = jnp.concatenate(
            [texp, tblock, pad], axis=0)[None, :].astype(jnp.int32)


MAX_TILES_PAD = 32
META_W = 128


def _router(hidden, gate_wt):
    grid = T // TILE
    return pl.pallas_call(
        _router_body,
        grid=(grid,),
        in_specs=[
            pl.BlockSpec((TILE, D_MODEL), lambda i: (i, 0)),
            pl.BlockSpec((D_MODEL, E), lambda i: (0, 0)),
        ],
        out_specs=[
            pl.BlockSpec((8, TILE), lambda i: (0, i)),
            pl.BlockSpec((TILE, 16), lambda i: (i, 0)),
            pl.BlockSpec((TILE, 16), lambda i: (i, 0)),
            pl.BlockSpec((1, META_W), lambda i: (0, 0)),
        ],
        out_shape=[
            jax.ShapeDtypeStruct((8, T), jnp.int32),
            jax.ShapeDtypeStruct((T, 16), jnp.float32),
            jax.ShapeDtypeStruct((T, 16), jnp.float32),
            jax.ShapeDtypeStruct((1, META_W), jnp.int32),
        ],
        scratch_shapes=[pltpu.VMEM((1, E), jnp.float32)],
    )(hidden, gate_wt)


def _shared_body(x_ref, sgu_ref, sdown_ref, out_ref):
    xb = x_ref[...].astype(jnp.bfloat16)
    h = jnp.dot(xb, sgu_ref[...], preferred_element_type=jnp.float32)
    g = h[:, :SHARED_FF]
    u = h[:, SHARED_FF:]
    act = (_silu(g) * u).astype(jnp.bfloat16)
    out_ref[...] = jnp.dot(act, sdown_ref[...],
                           preferred_element_type=jnp.float32)


def _shared(hidden, sgu, sdown):
    grid = T // TILE
    return pl.pallas_call(
        _shared_body,
        grid=(grid,),
        in_specs=[
            pl.BlockSpec((TILE, D_MODEL), lambda i: (i, 0)),
            pl.BlockSpec((D_MODEL, 2 * SHARED_FF), lambda i: (0, 0)),
            pl.BlockSpec((SHARED_FF, D_MODEL), lambda i: (0, 0)),
        ],
        out_specs=pl.BlockSpec((TILE, D_MODEL), lambda i: (i, 0)),
        out_shape=jax.ShapeDtypeStruct((T, D_MODEL), jnp.float32),
    )(hidden, sgu, sdown)


def _dispatch_body(hid_hbm, s0_hbm, s1_hbm, xs_hbm, hid_v, s0_v, s1_v, sem):
    wid = lax.axis_index("s") * 2 + lax.axis_index("c")
    base = wid * TOK_PER_W
    pltpu.sync_copy(hid_hbm.at[pl.ds(base, TOK_PER_W)], hid_v)
    pltpu.sync_copy(s0_hbm.at[pl.ds(base, TOK_PER_W)], s0_v)
    pltpu.sync_copy(s1_hbm.at[pl.ds(base, TOK_PER_W)], s1_v)
    c0 = pltpu.make_async_copy(hid_v, xs_hbm.at[s0_v], sem)
    c0.start()
    c1 = pltpu.make_async_copy(hid_v, xs_hbm.at[s1_v], sem)
    c1.start()
    c0.wait()
    c1.wait()


def _dispatch(hidden, s0, s1):
    mesh = plsc.VectorSubcoreMesh(core_axis_name="c", subcore_axis_name="s", num_cores=2, num_subcores=16)
    f = pl.kernel(
        _dispatch_body,
        out_type=jax.ShapeDtypeStruct((N_BUF, D_MODEL), jnp.float32),
        mesh=mesh,
        scratch_types=[
            pltpu.VMEM((TOK_PER_W, D_MODEL), jnp.float32),
            pltpu.VMEM((TOK_PER_W,), jnp.int32),
            pltpu.VMEM((TOK_PER_W,), jnp.int32),
            pltpu.SemaphoreType.DMA,
        ],
    )
    return f(hidden, s0, s1)


def _gmm_body(texp_ref, tblock_ref, x_ref, wgu_ref, wdown_ref, y_ref):
    i = pl.program_id(0)

    @pl.when(tblock_ref[i] != DUMP_BLOCK)
    def _():
        xb = x_ref[...].astype(jnp.bfloat16)
        h = jnp.dot(xb, wgu_ref[0], preferred_element_type=jnp.float32)
        g = h[:, :MOE_FF]
        u = h[:, MOE_FF:]
        act = (_silu(g) * u).astype(jnp.bfloat16)
        y_ref[...] = jnp.dot(act, wdown_ref[0],
                             preferred_element_type=jnp.float32)


def _gmm(texp, tblock, x_sorted, wgu, wdown):
    grid_spec = pltpu.PrefetchScalarGridSpec(
        num_scalar_prefetch=2,
        grid=(MAX_TILES,),
        in_specs=[
            pl.BlockSpec((TILE, D_MODEL), lambda i, texp, tblock: (tblock[i], 0)),
            pl.BlockSpec((1, D_MODEL, 2 * MOE_FF),
                         lambda i, texp, tblock: (texp[i], 0, 0)),
            pl.BlockSpec((1, MOE_FF, D_MODEL),
                         lambda i, texp, tblock: (texp[i], 0, 0)),
        ],
        out_specs=pl.BlockSpec((TILE, D_MODEL),
                               lambda i, texp, tblock: (tblock[i], 0)),
    )
    return pl.pallas_call(
        _gmm_body,
        grid_spec=grid_spec,
        out_shape=jax.ShapeDtypeStruct((N_BUF, D_MODEL), jnp.float32),
    )(texp, tblock, x_sorted, wgu, wdown)


def _combine_body(y_hbm, s0_hbm, s1_hbm, w0_hbm, w1_hbm, sh_hbm, out_hbm,
                  g0_v, g1_v, sh_v, o_v, s0_v, s1_v, w0_v, w1_v, sem):
    wid = lax.axis_index("s") * 2 + lax.axis_index("c")
    for chunk in range(TOK_PER_W // 16):
        base = wid * TOK_PER_W + chunk * 16
        pltpu.sync_copy(s0_hbm.at[pl.ds(base, 16)], s0_v)
        pltpu.sync_copy(s1_hbm.at[pl.ds(base, 16)], s1_v)
        pltpu.sync_copy(w0_hbm.at[pl.ds(base, 16)], w0_v)
        pltpu.sync_copy(w1_hbm.at[pl.ds(base, 16)], w1_v)
        pltpu.sync_copy(sh_hbm.at[pl.ds(base, 16)], sh_v)
        c0 = pltpu.make_async_copy(y_hbm.at[s0_v], g0_v, sem)
        c0.start()
        c1 = pltpu.make_async_copy(y_hbm.at[s1_v], g1_v, sem)
        c1.start()
        c0.wait()
        c1.wait()

        def tok_body(j, _):
            w0s = w0_v[j, :]
            w1s = w1_v[j, :]
            for c in range(D_MODEL // 16):
                sl = pl.ds(c * 16, 16)
                o_v[j, sl] = (sh_v[j, sl] + w0s * g0_v[j, sl]
                              + w1s * g1_v[j, sl])
            return 0

        lax.fori_loop(0, 16, tok_body, 0)
        pltpu.sync_copy(o_v, out_hbm.at[pl.ds(base, 16)])


def _combine(y, s0, s1, w0, w1, shared_out):
    mesh = plsc.VectorSubcoreMesh(core_axis_name="c", subcore_axis_name="s", num_cores=2, num_subcores=16)
    f = pl.kernel(
        _combine_body,
        out_type=jax.ShapeDtypeStruct((T, D_MODEL), jnp.float32),
        mesh=mesh,
        scratch_types=[
            pltpu.VMEM((16, D_MODEL), jnp.float32),
            pltpu.VMEM((16, D_MODEL), jnp.float32),
            pltpu.VMEM((16, D_MODEL), jnp.float32),
            pltpu.VMEM((16, D_MODEL), jnp.float32),
            pltpu.VMEM((16,), jnp.int32),
            pltpu.VMEM((16,), jnp.int32),
            pltpu.VMEM((16, 16), jnp.float32),
            pltpu.VMEM((16, 16), jnp.float32),
            pltpu.SemaphoreType.DMA,
        ],
    )
    return f(y, s0, s1, w0, w1, shared_out)


@jax.jit
def kernel(hidden_states, gate_w, w_gate_up, w_down, shared_gate_up,
           shared_down):
    gate_wt = gate_w.T
    sgu = shared_gate_up.astype(jnp.bfloat16)
    sdown = shared_down.astype(jnp.bfloat16)
    wgu = w_gate_up.astype(jnp.bfloat16)
    wdown = w_down.astype(jnp.bfloat16)

    slots, w0e, w1e, meta = _router(hidden_states, gate_wt)
    s0 = slots[0]
    s1 = slots[1]
    texp = meta[0, :MAX_TILES_PAD]
    tblock = meta[0, MAX_TILES_PAD:2 * MAX_TILES_PAD]

    shared_out = _shared(hidden_states, sgu, sdown)
    x_sorted = _dispatch(hidden_states, s0, s1)
    y = _gmm(texp, tblock, x_sorted, wgu, wdown)
    return _combine(y, s0, s1, w0e, w1e, shared_out)


# speedup vs baseline: 1.1475x; 1.1275x over previous
import functools

import jax
import jax.numpy as jnp
from jax import lax
from jax.experimental import pallas as pl
from jax.experimental.pallas import tpu as pltpu
from jax.experimental.pallas import tpu_sc as plsc

E = 8
TOP_K = 2
D_MODEL = 1024
MOE_FF = 512
SHARED_FF = 1024
T = 2048

TILE = 256
CAP = 2048
BLOCKS_PER_E = CAP // TILE
MAX_TILES = 24
MAX_TILES_PAD = 32
META_W = 128
DUMP_BLOCK = E * BLOCKS_PER_E
N_BUF = (DUMP_BLOCK + 1) * TILE

NW = 32
TOK_PER_W = T // NW
HALF_W = TOK_PER_W // 2
CHUNK = 16
N_CHUNK = TOK_PER_W // CHUNK


def _silu(x):
    return x * jax.nn.sigmoid(x)


def _head_body(x_ref, gate_wt_ref, sgu_ref, sdown_ref,
               sh_ref, slots_ref, w0e_ref, w1e_ref, meta_ref, run_ref):
    i = pl.program_id(0)
    x = x_ref[...]
    xb = x.astype(jnp.bfloat16)

    h = jnp.dot(xb, sgu_ref[...], preferred_element_type=jnp.float32)
    g = h[:, :SHARED_FF]
    u = h[:, SHARED_FF:]
    act = (_silu(g) * u).astype(jnp.bfloat16)
    sh_ref[...] = jnp.dot(act, sdown_ref[...],
                          preferred_element_type=jnp.float32)

    logits = lax.dot_general(
        x, gate_wt_ref[...], (((1,), (0,)), ((), ())),
        preferred_element_type=jnp.float32)
    m = jnp.max(logits, axis=-1, keepdims=True)
    ex = jnp.exp(logits - m)
    probs = ex / jnp.sum(ex, axis=-1, keepdims=True)

    iota = lax.broadcasted_iota(jnp.int32, probs.shape, 1)
    p1 = jnp.max(probs, axis=-1, keepdims=True)
    id1 = jnp.min(jnp.where(probs == p1, iota, E), axis=-1, keepdims=True)
    oh1 = iota == id1
    probs_m = jnp.where(oh1, -jnp.inf, probs)
    p2 = jnp.max(probs_m, axis=-1, keepdims=True)
    id2 = jnp.min(jnp.where(probs_m == p2, iota, E), axis=-1, keepdims=True)
    oh2 = iota == id2
    denom = p1 + p2
    w0e_ref[...] = jnp.broadcast_to(p1 / denom, (TILE, 16))
    w1e_ref[...] = jnp.broadcast_to(p2 / denom, (TILE, 16))

    a = (oh1 | oh2).astype(jnp.float32)

    ri = lax.broadcasted_iota(jnp.int32, (TILE, TILE), 0)
    ci = lax.broadcasted_iota(jnp.int32, (TILE, TILE), 1)
    tri = (ci < ri).astype(jnp.bfloat16)
    excl = lax.dot_general(
        tri, a.astype(jnp.bfloat16), (((1,), (0,)), ((), ())),
        preferred_element_type=jnp.float32)

    @pl.when(i == 0)
    def _():
        run_ref[...] = jnp.zeros_like(run_ref)

    run = run_ref[...]
    pos = excl + run
    new_run = run + jnp.sum(a, axis=0, keepdims=True)
    run_ref[...] = new_run

    eoff = lax.broadcasted_iota(jnp.int32, (TILE, E), 1).astype(
        jnp.float32) * CAP
    slotmat = pos + eoff
    slot0 = jnp.sum(jnp.where(oh1, slotmat, 0.0), axis=1)
    slot1 = jnp.sum(jnp.where(oh2, slotmat, 0.0), axis=1)

    pad6 = jnp.zeros((6, TILE), jnp.float32)
    slots_ref[...] = jnp.concatenate(
        [slot0[None, :], slot1[None, :], pad6], axis=0).astype(jnp.int32)

    @pl.when(i == pl.num_programs(0) - 1)
    def _():
        counts = new_run
        ntiles = jnp.floor((counts + (TILE - 1)) / TILE)
        e1 = lax.broadcasted_iota(jnp.int32, (E, E), 0)
        e2 = lax.broadcasted_iota(jnp.int32, (E, E), 1)
        strict = (e1 < e2).astype(jnp.float32)
        cum_excl = lax.dot_general(
            ntiles, strict, (((1,), (0,)), ((), ())),
            preferred_element_type=jnp.float32)
        nvalid = jnp.sum(ntiles, axis=-1, keepdims=True)

        ti = lax.broadcasted_iota(jnp.int32, (MAX_TILES_PAD, E), 0).astype(
            jnp.float32)
        ge = ti >= cum_excl
        texp = jnp.sum(ge.astype(jnp.float32), axis=1) - 1.0
        base = jnp.max(jnp.where(ge, jnp.broadcast_to(cum_excl, ge.shape),
                                 -1.0), axis=1)
        icol = lax.broadcasted_iota(jnp.int32, (MAX_TILES_PAD,), 0).astype(
            jnp.float32)
        tblock = texp * BLOCKS_PER_E + (icol - base)
        invalid = icol >= nvalid[0, 0]
        tblock = jnp.where(invalid, float(DUMP_BLOCK), tblock)
        texp = jnp.where(invalid, 0.0, texp)
        pad = jnp.zeros((META_W - 2 * MAX_TILES_PAD,), jnp.float32)
        meta_ref[...] = jnp.concatenate(
            [texp, tblock, pad], axis=0)[None, :].astype(jnp.int32)


def _head(hidden, gate_wt, sgu, sdown):
    grid = T // TILE
    return pl.pallas_call(
        _head_body,
        grid=(grid,),
        in_specs=[
            pl.BlockSpec((TILE, D_MODEL), lambda i: (i, 0)),
            pl.BlockSpec((D_MODEL, E), lambda i: (0, 0)),
            pl.BlockSpec((D_MODEL, 2 * SHARED_FF), lambda i: (0, 0)),
            pl.BlockSpec((SHARED_FF, D_MODEL), lambda i: (0, 0)),
        ],
        out_specs=[
            pl.BlockSpec((TILE, D_MODEL), lambda i: (i, 0)),
            pl.BlockSpec((8, TILE), lambda i: (0, i)),
            pl.BlockSpec((TILE, 16), lambda i: (i, 0)),
            pl.BlockSpec((TILE, 16), lambda i: (i, 0)),
            pl.BlockSpec((1, META_W), lambda i: (0, 0)),
        ],
        out_shape=[
            jax.ShapeDtypeStruct((T, D_MODEL), jnp.float32),
            jax.ShapeDtypeStruct((8, T), jnp.int32),
            jax.ShapeDtypeStruct((T, 16), jnp.float32),
            jax.ShapeDtypeStruct((T, 16), jnp.float32),
            jax.ShapeDtypeStruct((1, META_W), jnp.int32),
        ],
        scratch_shapes=[pltpu.VMEM((1, E), jnp.float32)],
    )(hidden, gate_wt, sgu, sdown)


def _dispatch_body(hid_hbm, s0_hbm, s1_hbm, xs_hbm,
                   hid_a, hid_b, s0a, s0b, s1a, s1b, sem_r, sem_w):
    wid = lax.axis_index("s") * 2 + lax.axis_index("c")
    base = wid * TOK_PER_W
    ra = pltpu.make_async_copy(hid_hbm.at[pl.ds(base, HALF_W)], hid_a, sem_r)
    ra.start()
    rb = pltpu.make_async_copy(
        hid_hbm.at[pl.ds(base + HALF_W, HALF_W)], hid_b, sem_r)
    rb.start()
    pltpu.sync_copy(s0_hbm.at[pl.ds(base, HALF_W)], s0a)
    pltpu.sync_copy(s0_hbm.at[pl.ds(base + HALF_W, HALF_W)], s0b)
    pltpu.sync_copy(s1_hbm.at[pl.ds(base, HALF_W)], s1a)
    pltpu.sync_copy(s1_hbm.at[pl.ds(base + HALF_W, HALF_W)], s1b)
    ra.wait()
    wa0 = pltpu.make_async_copy(hid_a, xs_hbm.at[s0a], sem_w)
    wa0.start()
    wa1 = pltpu.make_async_copy(hid_a, xs_hbm.at[s1a], sem_w)
    wa1.start()
    rb.wait()
    wb0 = pltpu.make_async_copy(hid_b, xs_hbm.at[s0b], sem_w)
    wb0.start()
    wb1 = pltpu.make_async_copy(hid_b, xs_hbm.at[s1b], sem_w)
    wb1.start()
    wa0.wait()
    wa1.wait()
    wb0.wait()
    wb1.wait()


def _dispatch(hidden, s0, s1):
    mesh = plsc.VectorSubcoreMesh(core_axis_name="c", subcore_axis_name="s",
                                  num_cores=2, num_subcores=16)
    f = pl.kernel(
        _dispatch_body,
        out_type=jax.ShapeDtypeStruct((N_BUF, D_MODEL), jnp.float32),
        mesh=mesh,
        scratch_types=[
            pltpu.VMEM((HALF_W, D_MODEL), jnp.float32),
            pltpu.VMEM((HALF_W, D_MODEL), jnp.float32),
            pltpu.VMEM((HALF_W,), jnp.int32),
            pltpu.VMEM((HALF_W,), jnp.int32),
            pltpu.VMEM((HALF_W,), jnp.int32),
            pltpu.VMEM((HALF_W,), jnp.int32),
            pltpu.SemaphoreType.DMA,
            pltpu.SemaphoreType.DMA,
        ],
    )
    return f(hidden, s0, s1)


def _gmm_body(texp_ref, tblock_ref, x_ref, wgu_ref, wdown_ref, y_ref):
    i = pl.program_id(0)

    @pl.when(tblock_ref[i] != DUMP_BLOCK)
    def _():
        xb = x_ref[...].astype(jnp.bfloat16)
        h = jnp.dot(xb, wgu_ref[0], preferred_element_type=jnp.float32)
        g = h[:, :MOE_FF]
        u = h[:, MOE_FF:]
        act = (_silu(g) * u).astype(jnp.bfloat16)
        y_ref[...] = jnp.dot(act, wdown_ref[0],
                             preferred_element_type=jnp.float32)


def _gmm(texp, tblock, x_sorted, wgu, wdown):
    grid_spec = pltpu.PrefetchScalarGridSpec(
        num_scalar_prefetch=2,
        grid=(MAX_TILES,),
        in_specs=[
            pl.BlockSpec((TILE, D_MODEL),
                         lambda i, texp, tblock: (tblock[i], 0)),
            pl.BlockSpec((1, D_MODEL, 2 * MOE_FF),
                         lambda i, texp, tblock: (texp[i], 0, 0)),
            pl.BlockSpec((1, MOE_FF, D_MODEL),
                         lambda i, texp, tblock: (texp[i], 0, 0)),
        ],
        out_specs=pl.BlockSpec((TILE, D_MODEL),
                               lambda i, texp, tblock: (tblock[i], 0)),
    )
    return pl.pallas_call(
        _gmm_body,
        grid_spec=grid_spec,
        out_shape=jax.ShapeDtypeStruct((N_BUF, D_MODEL), jnp.float32),
    )(texp, tblock, x_sorted, wgu, wdown)


def _combine_body(y_hbm, s0_hbm, s1_hbm, w0_hbm, w1_hbm, sh_hbm, out_hbm,
                  g0_a, g0_b, g1_a, g1_b, sh_a, sh_b,
                  s0_v, s1_v, w0_v, w1_v, sem_a, sem_b, sem_o):
    wid = lax.axis_index("s") * 2 + lax.axis_index("c")
    base = wid * TOK_PER_W
    pltpu.sync_copy(s0_hbm.at[pl.ds(base, TOK_PER_W)], s0_v)
    pltpu.sync_copy(s1_hbm.at[pl.ds(base, TOK_PER_W)], s1_v)
    pltpu.sync_copy(w0_hbm.at[pl.ds(base, TOK_PER_W)], w0_v)
    pltpu.sync_copy(w1_hbm.at[pl.ds(base, TOK_PER_W)], w1_v)

    bufs = ((g0_a, g1_a, sh_a, sem_a), (g0_b, g1_b, sh_b, sem_b))
    out_pend = [None, None]

    def start(chunk):
        g0, g1, sh, sem = bufs[chunk % 2]
        if out_pend[chunk % 2] is not None:
            out_pend[chunk % 2].wait()
            out_pend[chunk % 2] = None
        idx0 = s0_v[pl.ds(chunk * CHUNK, CHUNK)]
        idx1 = s1_v[pl.ds(chunk * CHUNK, CHUNK)]
        cs = (pltpu.make_async_copy(y_hbm.at[idx0], g0, sem),
              pltpu.make_async_copy(y_hbm.at[idx1], g1, sem),
              pltpu.make_async_copy(
                  sh_hbm.at[pl.ds(base + chunk * CHUNK, CHUNK)], sh, sem))
        for c in cs:
            c.start()
        return cs

    pend = start(0)
    for chunk in range(N_CHUNK):
        g0, g1, sh, sem = bufs[chunk % 2]
        for c in pend:
            c.wait()
        if chunk + 1 < N_CHUNK:
            pend = start(chunk + 1)

        def tok(j, _):
            w0s = w0_v[j + chunk * CHUNK, :]
            w1s = w1_v[j + chunk * CHUNK, :]
            for c in range(D_MODEL // 16):
                sl = pl.ds(c * 16, 16)
                sh[j, sl] = sh[j, sl] + w0s * g0[j, sl] + w1s * g1[j, sl]
            return 0

        lax.fori_loop(0, CHUNK, tok, 0)
        oc = pltpu.make_async_copy(
            sh, out_hbm.at[pl.ds(base + chunk * CHUNK, CHUNK)], sem_o)
        oc.start()
        out_pend[chunk % 2] = oc
    for oc in out_pend:
        if oc is not None:
            oc.wait()


def _combine(y, s0, s1, w0e, w1e, shared_out):
    mesh = plsc.VectorSubcoreMesh(core_axis_name="c", subcore_axis_name="s",
                                  num_cores=2, num_subcores=16)
    f = pl.kernel(
        _combine_body,
        out_type=jax.ShapeDtypeStruct((T, D_MODEL), jnp.float32),
        mesh=mesh,
        scratch_types=[
            pltpu.VMEM((CHUNK, D_MODEL), jnp.float32),
            pltpu.VMEM((CHUNK, D_MODEL), jnp.float32),
            pltpu.VMEM((CHUNK, D_MODEL), jnp.float32),
            pltpu.VMEM((CHUNK, D_MODEL), jnp.float32),
            pltpu.VMEM((CHUNK, D_MODEL), jnp.float32),
            pltpu.VMEM((CHUNK, D_MODEL), jnp.float32),
            pltpu.VMEM((TOK_PER_W,), jnp.int32),
            pltpu.VMEM((TOK_PER_W,), jnp.int32),
            pltpu.VMEM((TOK_PER_W, 16), jnp.float32),
            pltpu.VMEM((TOK_PER_W, 16), jnp.float32),
            pltpu.SemaphoreType.DMA,
            pltpu.SemaphoreType.DMA,
            pltpu.SemaphoreType.DMA,
        ],
    )
    return f(y, s0, s1, w0e, w1e, shared_out)


@jax.jit
def kernel(hidden_states, gate_w, w_gate_up, w_down, shared_gate_up,
           shared_down):
    gate_wt = gate_w.T
    sgu = shared_gate_up.astype(jnp.bfloat16)
    sdown = shared_down.astype(jnp.bfloat16)
    wgu = w_gate_up.astype(jnp.bfloat16)
    wdown = w_down.astype(jnp.bfloat16)

    shared_out, slots, w0e, w1e, meta = _head(hidden_states, gate_wt, sgu,
                                              sdown)
    s0 = slots[0]
    s1 = slots[1]
    texp = meta[0, :MAX_TILES_PAD]
    tblock = meta[0, MAX_TILES_PAD:2 * MAX_TILES_PAD]

    x_sorted = _dispatch(hidden_states, s0, s1)
    y = _gmm(texp, tblock, x_sorted, wgu, wdown)
    return _combine(y, s0, s1, w0e, w1e, shared_out)


# speedup vs baseline: 1.3360x; 1.1643x over previous
import functools

import jax
import jax.numpy as jnp
from jax import lax
from jax.experimental import pallas as pl
from jax.experimental.pallas import tpu as pltpu
from jax.experimental.pallas import tpu_sc as plsc

E = 8
TOP_K = 2
D_MODEL = 1024
MOE_FF = 512
SHARED_FF = 1024
T = 2048

TILE = 256
CAP = 2048
BLOCKS_PER_E = CAP // TILE
MAX_TILES = 24
MAX_TILES_PAD = 32
META_W = 128
DUMP_BLOCK = E * BLOCKS_PER_E
N_BUF = (DUMP_BLOCK + 1) * TILE

NW = 32
TOK_PER_W = T // NW
HALF_W = TOK_PER_W // 2
CHUNK = 16
N_CHUNK = TOK_PER_W // CHUNK


def _silu(x):
    return x * jax.nn.sigmoid(x)


def _head_body(x_ref, gate_wt_ref, sgu_ref, sdown_ref,
               sh_ref, slots_ref, w0e_ref, w1e_ref, meta_ref, run_ref):
    i = pl.program_id(0)
    x = x_ref[...]
    xb = x.astype(jnp.bfloat16)

    h = jnp.dot(xb, sgu_ref[...].astype(jnp.bfloat16),
                preferred_element_type=jnp.float32)
    g = h[:, :SHARED_FF]
    u = h[:, SHARED_FF:]
    act = (_silu(g) * u).astype(jnp.bfloat16)
    sh_ref[...] = jnp.dot(act, sdown_ref[...].astype(jnp.bfloat16),
                          preferred_element_type=jnp.float32)

    logits = lax.dot_general(
        x, gate_wt_ref[...], (((1,), (0,)), ((), ())),
        preferred_element_type=jnp.float32)
    m = jnp.max(logits, axis=-1, keepdims=True)
    ex = jnp.exp(logits - m)
    probs = ex / jnp.sum(ex, axis=-1, keepdims=True)

    iota = lax.broadcasted_iota(jnp.int32, probs.shape, 1)
    p1 = jnp.max(probs, axis=-1, keepdims=True)
    id1 = jnp.min(jnp.where(probs == p1, iota, E), axis=-1, keepdims=True)
    oh1 = iota == id1
    probs_m = jnp.where(oh1, -jnp.inf, probs)
    p2 = jnp.max(probs_m, axis=-1, keepdims=True)
    id2 = jnp.min(jnp.where(probs_m == p2, iota, E), axis=-1, keepdims=True)
    oh2 = iota == id2
    denom = p1 + p2
    w0e_ref[...] = jnp.broadcast_to(p1 / denom, (TILE, 16))
    w1e_ref[...] = jnp.broadcast_to(p2 / denom, (TILE, 16))

    a = (oh1 | oh2).astype(jnp.float32)

    ri = lax.broadcasted_iota(jnp.int32, (TILE, TILE), 0)
    ci = lax.broadcasted_iota(jnp.int32, (TILE, TILE), 1)
    tri = (ci < ri).astype(jnp.bfloat16)
    excl = lax.dot_general(
        tri, a.astype(jnp.bfloat16), (((1,), (0,)), ((), ())),
        preferred_element_type=jnp.float32)

    @pl.when(i == 0)
    def _():
        run_ref[...] = jnp.zeros_like(run_ref)

    run = run_ref[...]
    pos = excl + run
    new_run = run + jnp.sum(a, axis=0, keepdims=True)
    run_ref[...] = new_run

    eoff = lax.broadcasted_iota(jnp.int32, (TILE, E), 1).astype(
        jnp.float32) * CAP
    slotmat = pos + eoff
    slot0 = jnp.sum(jnp.where(oh1, slotmat, 0.0), axis=1)
    slot1 = jnp.sum(jnp.where(oh2, slotmat, 0.0), axis=1)

    pad6 = jnp.zeros((6, TILE), jnp.float32)
    slots_ref[...] = jnp.concatenate(
        [slot0[None, :], slot1[None, :], pad6], axis=0).astype(jnp.int32)

    @pl.when(i == pl.num_programs(0) - 1)
    def _():
        counts = new_run
        ntiles = jnp.floor((counts + (TILE - 1)) / TILE)
        e1 = lax.broadcasted_iota(jnp.int32, (E, E), 0)
        e2 = lax.broadcasted_iota(jnp.int32, (E, E), 1)
        strict = (e1 < e2).astype(jnp.float32)
        cum_excl = lax.dot_general(
            ntiles, strict, (((1,), (0,)), ((), ())),
            preferred_element_type=jnp.float32)
        nvalid = jnp.sum(ntiles, axis=-1, keepdims=True)

        ti = lax.broadcasted_iota(jnp.int32, (MAX_TILES_PAD, E), 0).astype(
            jnp.float32)
        ge = ti >= cum_excl
        texp = jnp.sum(ge.astype(jnp.float32), axis=1) - 1.0
        base = jnp.max(jnp.where(ge, jnp.broadcast_to(cum_excl, ge.shape),
                                 -1.0), axis=1)
        icol = lax.broadcasted_iota(jnp.int32, (MAX_TILES_PAD,), 0).astype(
            jnp.float32)
        tblock = texp * BLOCKS_PER_E + (icol - base)
        invalid = icol >= nvalid[0, 0]
        tblock = jnp.where(invalid, float(DUMP_BLOCK), tblock)
        texp = jnp.where(invalid, 0.0, texp)
        pad = jnp.zeros((META_W - 2 * MAX_TILES_PAD,), jnp.float32)
        meta_ref[...] = jnp.concatenate(
            [texp, tblock, pad], axis=0)[None, :].astype(jnp.int32)


def _head(hidden, gate_wt, sgu, sdown):
    grid = T // TILE
    return pl.pallas_call(
        _head_body,
        grid=(grid,),
        in_specs=[
            pl.BlockSpec((TILE, D_MODEL), lambda i: (i, 0)),
            pl.BlockSpec((D_MODEL, E), lambda i: (0, 0)),
            pl.BlockSpec((D_MODEL, 2 * SHARED_FF), lambda i: (0, 0)),
            pl.BlockSpec((SHARED_FF, D_MODEL), lambda i: (0, 0)),
        ],
        out_specs=[
            pl.BlockSpec((TILE, D_MODEL), lambda i: (i, 0)),
            pl.BlockSpec((8, TILE), lambda i: (0, i)),
            pl.BlockSpec((TILE, 16), lambda i: (i, 0)),
            pl.BlockSpec((TILE, 16), lambda i: (i, 0)),
            pl.BlockSpec((1, META_W), lambda i: (0, 0)),
        ],
        out_shape=[
            jax.ShapeDtypeStruct((T, D_MODEL), jnp.float32),
            jax.ShapeDtypeStruct((8, T), jnp.int32),
            jax.ShapeDtypeStruct((T, 16), jnp.float32),
            jax.ShapeDtypeStruct((T, 16), jnp.float32),
            jax.ShapeDtypeStruct((1, META_W), jnp.int32),
        ],
        scratch_shapes=[pltpu.VMEM((1, E), jnp.float32)],
    )(hidden, gate_wt, sgu, sdown)


def _dispatch_body(hid_hbm, s0_hbm, s1_hbm, xs_hbm,
                   hid_a, hid_b, s0a, s0b, s1a, s1b, sem_r, sem_w):
    wid = lax.axis_index("s") * 2 + lax.axis_index("c")
    base = wid * TOK_PER_W
    ra = pltpu.make_async_copy(hid_hbm.at[pl.ds(base, HALF_W)], hid_a, sem_r)
    ra.start()
    rb = pltpu.make_async_copy(
        hid_hbm.at[pl.ds(base + HALF_W, HALF_W)], hid_b, sem_r)
    rb.start()
    pltpu.sync_copy(s0_hbm.at[pl.ds(base, HALF_W)], s0a)
    pltpu.sync_copy(s0_hbm.at[pl.ds(base + HALF_W, HALF_W)], s0b)
    pltpu.sync_copy(s1_hbm.at[pl.ds(base, HALF_W)], s1a)
    pltpu.sync_copy(s1_hbm.at[pl.ds(base + HALF_W, HALF_W)], s1b)
    ra.wait()
    wa0 = pltpu.make_async_copy(hid_a, xs_hbm.at[s0a], sem_w)
    wa0.start()
    wa1 = pltpu.make_async_copy(hid_a, xs_hbm.at[s1a], sem_w)
    wa1.start()
    rb.wait()
    wb0 = pltpu.make_async_copy(hid_b, xs_hbm.at[s0b], sem_w)
    wb0.start()
    wb1 = pltpu.make_async_copy(hid_b, xs_hbm.at[s1b], sem_w)
    wb1.start()
    wa0.wait()
    wa1.wait()
    wb0.wait()
    wb1.wait()


def _dispatch(hidden, s0, s1):
    mesh = plsc.VectorSubcoreMesh(core_axis_name="c", subcore_axis_name="s",
                                  num_cores=2, num_subcores=16)
    f = pl.kernel(
        _dispatch_body,
        out_type=jax.ShapeDtypeStruct((N_BUF, D_MODEL), jnp.float32),
        mesh=mesh,
        scratch_types=[
            pltpu.VMEM((HALF_W, D_MODEL), jnp.float32),
            pltpu.VMEM((HALF_W, D_MODEL), jnp.float32),
            pltpu.VMEM((HALF_W,), jnp.int32),
            pltpu.VMEM((HALF_W,), jnp.int32),
            pltpu.VMEM((HALF_W,), jnp.int32),
            pltpu.VMEM((HALF_W,), jnp.int32),
            pltpu.SemaphoreType.DMA,
            pltpu.SemaphoreType.DMA,
        ],
    )
    return f(hidden, s0, s1)


def _gmm_body(texp_ref, tblock_ref, x_ref, wgu_ref, wdown_ref, y_ref):
    i = pl.program_id(0)

    @pl.when(tblock_ref[i] != DUMP_BLOCK)
    def _():
        xb = x_ref[...].astype(jnp.bfloat16)
        h = jnp.dot(xb, wgu_ref[0].astype(jnp.bfloat16),
                    preferred_element_type=jnp.float32)
        g = h[:, :MOE_FF]
        u = h[:, MOE_FF:]
        act = (_silu(g) * u).astype(jnp.bfloat16)
        y_ref[...] = jnp.dot(act, wdown_ref[0].astype(jnp.bfloat16),
                             preferred_element_type=jnp.float32)


def _gmm(texp, tblock, x_sorted, wgu, wdown):
    grid_spec = pltpu.PrefetchScalarGridSpec(
        num_scalar_prefetch=2,
        grid=(MAX_TILES,),
        in_specs=[
            pl.BlockSpec((TILE, D_MODEL),
                         lambda i, texp, tblock: (tblock[i], 0)),
            pl.BlockSpec((1, D_MODEL, 2 * MOE_FF),
                         lambda i, texp, tblock: (texp[i], 0, 0)),
            pl.BlockSpec((1, MOE_FF, D_MODEL),
                         lambda i, texp, tblock: (texp[i], 0, 0)),
        ],
        out_specs=pl.BlockSpec((TILE, D_MODEL),
                               lambda i, texp, tblock: (tblock[i], 0)),
    )
    return pl.pallas_call(
        _gmm_body,
        grid_spec=grid_spec,
        out_shape=jax.ShapeDtypeStruct((N_BUF, D_MODEL), jnp.float32),
    )(texp, tblock, x_sorted, wgu, wdown)


def _combine_body(y_hbm, s0_hbm, s1_hbm, w0_hbm, w1_hbm, sh_hbm, out_hbm,
                  g0_a, g0_b, g1_a, g1_b, sh_a, sh_b,
                  s0_v, s1_v, w0_v, w1_v, sem_a, sem_b, sem_o):
    wid = lax.axis_index("s") * 2 + lax.axis_index("c")
    base = wid * TOK_PER_W
    pltpu.sync_copy(s0_hbm.at[pl.ds(base, TOK_PER_W)], s0_v)
    pltpu.sync_copy(s1_hbm.at[pl.ds(base, TOK_PER_W)], s1_v)
    pltpu.sync_copy(w0_hbm.at[pl.ds(base, TOK_PER_W)], w0_v)
    pltpu.sync_copy(w1_hbm.at[pl.ds(base, TOK_PER_W)], w1_v)

    bufs = ((g0_a, g1_a, sh_a, sem_a), (g0_b, g1_b, sh_b, sem_b))
    out_pend = [None, None]

    def start(chunk):
        g0, g1, sh, sem = bufs[chunk % 2]
        if out_pend[chunk % 2] is not None:
            out_pend[chunk % 2].wait()
            out_pend[chunk % 2] = None
        idx0 = s0_v[pl.ds(chunk * CHUNK, CHUNK)]
        idx1 = s1_v[pl.ds(chunk * CHUNK, CHUNK)]
        cs = (pltpu.make_async_copy(y_hbm.at[idx0], g0, sem),
              pltpu.make_async_copy(y_hbm.at[idx1], g1, sem),
              pltpu.make_async_copy(
                  sh_hbm.at[pl.ds(base + chunk * CHUNK, CHUNK)], sh, sem))
        for c in cs:
            c.start()
        return cs

    pend = start(0)
    for chunk in range(N_CHUNK):
        g0, g1, sh, sem = bufs[chunk % 2]
        for c in pend:
            c.wait()
        if chunk + 1 < N_CHUNK:
            pend = start(chunk + 1)

        def tok(j, _):
            w0s = w0_v[j + chunk * CHUNK, :]
            w1s = w1_v[j + chunk * CHUNK, :]
            for c in range(D_MODEL // 16):
                sl = pl.ds(c * 16, 16)
                sh[j, sl] = sh[j, sl] + w0s * g0[j, sl] + w1s * g1[j, sl]
            return 0

        lax.fori_loop(0, CHUNK, tok, 0)
        oc = pltpu.make_async_copy(
            sh, out_hbm.at[pl.ds(base + chunk * CHUNK, CHUNK)], sem_o)
        oc.start()
        out_pend[chunk % 2] = oc
    for oc in out_pend:
        if oc is not None:
            oc.wait()


def _combine(y, s0, s1, w0e, w1e, shared_out):
    mesh = plsc.VectorSubcoreMesh(core_axis_name="c", subcore_axis_name="s",
                                  num_cores=2, num_subcores=16)
    f = pl.kernel(
        _combine_body,
        out_type=jax.ShapeDtypeStruct((T, D_MODEL), jnp.float32),
        mesh=mesh,
        scratch_types=[
            pltpu.VMEM((CHUNK, D_MODEL), jnp.float32),
            pltpu.VMEM((CHUNK, D_MODEL), jnp.float32),
            pltpu.VMEM((CHUNK, D_MODEL), jnp.float32),
            pltpu.VMEM((CHUNK, D_MODEL), jnp.float32),
            pltpu.VMEM((CHUNK, D_MODEL), jnp.float32),
            pltpu.VMEM((CHUNK, D_MODEL), jnp.float32),
            pltpu.VMEM((TOK_PER_W,), jnp.int32),
            pltpu.VMEM((TOK_PER_W,), jnp.int32),
            pltpu.VMEM((TOK_PER_W, 16), jnp.float32),
            pltpu.VMEM((TOK_PER_W, 16), jnp.float32),
            pltpu.SemaphoreType.DMA,
            pltpu.SemaphoreType.DMA,
            pltpu.SemaphoreType.DMA,
        ],
    )
    return f(y, s0, s1, w0e, w1e, shared_out)


@jax.jit
def kernel(hidden_states, gate_w, w_gate_up, w_down, shared_gate_up,
           shared_down):
    gate_wt = gate_w.T

    shared_out, slots, w0e, w1e, meta = _head(hidden_states, gate_wt,
                                              shared_gate_up, shared_down)
    s0 = slots[0]
    s1 = slots[1]
    texp = meta[0, :MAX_TILES_PAD]
    tblock = meta[0, MAX_TILES_PAD:2 * MAX_TILES_PAD]

    x_sorted = _dispatch(hidden_states, s0, s1)
    y = _gmm(texp, tblock, x_sorted, w_gate_up, w_down)
    return _combine(y, s0, s1, w0e, w1e, shared_out)


# speedup vs baseline: 1.3571x; 1.0158x over previous
import functools

import jax
import jax.numpy as jnp
from jax import lax
from jax.experimental import pallas as pl
from jax.experimental.pallas import tpu as pltpu
from jax.experimental.pallas import tpu_sc as plsc

E = 8
TOP_K = 2
D_MODEL = 1024
MOE_FF = 512
SHARED_FF = 1024
T = 2048

TILE = 256
CAP = 2048
BLOCKS_PER_E = CAP // TILE
MAX_TILES = 24
MAX_TILES_PAD = 32
META_W = 128
DUMP_BLOCK = E * BLOCKS_PER_E
N_BUF = (DUMP_BLOCK + 1) * TILE

NW = 32
TOK_PER_W = T // NW
HALF_W = TOK_PER_W // 2
CHUNK = 16
N_CHUNK = TOK_PER_W // CHUNK


def _silu(x):
    return x * jax.nn.sigmoid(x)


def _pack(xb):
    lo = jax.lax.bitcast_convert_type(xb[:, :D_MODEL // 2], jnp.int16)
    hi = jax.lax.bitcast_convert_type(xb[:, D_MODEL // 2:], jnp.int16)
    return (hi.astype(jnp.int32) << 16) | (lo.astype(jnp.int32) & 0xFFFF)


def _unpack(v):
    lo = jax.lax.bitcast_convert_type(v.astype(jnp.int16), jnp.bfloat16)
    hi = jax.lax.bitcast_convert_type((v >> 16).astype(jnp.int16),
                                      jnp.bfloat16)
    return jnp.concatenate([lo, hi], axis=1)


def _head_body(x_ref, gate_wt_ref, sgu_ref, sdown_ref,
               sh_ref, slots_ref, xpk_ref, w0e_ref, w1e_ref, meta_ref,
               run_ref):
    i = pl.program_id(0)
    x = x_ref[...]
    xb = x.astype(jnp.bfloat16)
    xpk_ref[...] = _pack(xb)

    h = jnp.dot(xb, sgu_ref[...].astype(jnp.bfloat16),
                preferred_element_type=jnp.float32)
    g = h[:, :SHARED_FF]
    u = h[:, SHARED_FF:]
    act = (_silu(g) * u).astype(jnp.bfloat16)
    sh_ref[...] = jnp.dot(act, sdown_ref[...].astype(jnp.bfloat16),
                          preferred_element_type=jnp.float32)

    logits = lax.dot_general(
        x, gate_wt_ref[...], (((1,), (0,)), ((), ())),
        preferred_element_type=jnp.float32)
    m = jnp.max(logits, axis=-1, keepdims=True)
    ex = jnp.exp(logits - m)
    probs = ex / jnp.sum(ex, axis=-1, keepdims=True)

    iota = lax.broadcasted_iota(jnp.int32, probs.shape, 1)
    p1 = jnp.max(probs, axis=-1, keepdims=True)
    id1 = jnp.min(jnp.where(probs == p1, iota, E), axis=-1, keepdims=True)
    oh1 = iota == id1
    probs_m = jnp.where(oh1, -jnp.inf, probs)
    p2 = jnp.max(probs_m, axis=-1, keepdims=True)
    id2 = jnp.min(jnp.where(probs_m == p2, iota, E), axis=-1, keepdims=True)
    oh2 = iota == id2
    denom = p1 + p2
    w0e_ref[...] = jnp.broadcast_to(p1 / denom, (TILE, 16))
    w1e_ref[...] = jnp.broadcast_to(p2 / denom, (TILE, 16))

    a = (oh1 | oh2).astype(jnp.float32)

    ri = lax.broadcasted_iota(jnp.int32, (TILE, TILE), 0)
    ci = lax.broadcasted_iota(jnp.int32, (TILE, TILE), 1)
    tri = (ci < ri).astype(jnp.bfloat16)
    excl = lax.dot_general(
        tri, a.astype(jnp.bfloat16), (((1,), (0,)), ((), ())),
        preferred_element_type=jnp.float32)

    @pl.when(i == 0)
    def _():
        run_ref[...] = jnp.zeros_like(run_ref)

    run = run_ref[...]
    pos = excl + run
    new_run = run + jnp.sum(a, axis=0, keepdims=True)
    run_ref[...] = new_run

    eoff = lax.broadcasted_iota(jnp.int32, (TILE, E), 1).astype(
        jnp.float32) * CAP
    slotmat = pos + eoff
    slot0 = jnp.sum(jnp.where(oh1, slotmat, 0.0), axis=1)
    slot1 = jnp.sum(jnp.where(oh2, slotmat, 0.0), axis=1)

    pad6 = jnp.zeros((6, TILE), jnp.float32)
    slots_ref[...] = jnp.concatenate(
        [slot0[None, :], slot1[None, :], pad6], axis=0).astype(jnp.int32)

    @pl.when(i == pl.num_programs(0) - 1)
    def _():
        counts = new_run
        ntiles = jnp.floor((counts + (TILE - 1)) / TILE)
        e1 = lax.broadcasted_iota(jnp.int32, (E, E), 0)
        e2 = lax.broadcasted_iota(jnp.int32, (E, E), 1)
        strict = (e1 < e2).astype(jnp.float32)
        cum_excl = lax.dot_general(
            ntiles, strict, (((1,), (0,)), ((), ())),
            preferred_element_type=jnp.float32)
        nvalid = jnp.sum(ntiles, axis=-1, keepdims=True)

        ti = lax.broadcasted_iota(jnp.int32, (MAX_TILES_PAD, E), 0).astype(
            jnp.float32)
        ge = ti >= cum_excl
        texp = jnp.sum(ge.astype(jnp.float32), axis=1) - 1.0
        base = jnp.max(jnp.where(ge, jnp.broadcast_to(cum_excl, ge.shape),
                                 -1.0), axis=1)
        icol = lax.broadcasted_iota(jnp.int32, (MAX_TILES_PAD,), 0).astype(
            jnp.float32)
        tblock = texp * BLOCKS_PER_E + (icol - base)
        invalid = icol >= nvalid[0, 0]
        tblock = jnp.where(invalid, float(DUMP_BLOCK), tblock)
        texp = jnp.where(invalid, 0.0, texp)
        pad = jnp.zeros((META_W - 2 * MAX_TILES_PAD,), jnp.float32)
        meta_ref[...] = jnp.concatenate(
            [texp, tblock, pad], axis=0)[None, :].astype(jnp.int32)


def _head(hidden, gate_wt, sgu, sdown):
    grid = T // TILE
    return pl.pallas_call(
        _head_body,
        grid=(grid,),
        in_specs=[
            pl.BlockSpec((TILE, D_MODEL), lambda i: (i, 0)),
            pl.BlockSpec((D_MODEL, E), lambda i: (0, 0)),
            pl.BlockSpec((D_MODEL, 2 * SHARED_FF), lambda i: (0, 0)),
            pl.BlockSpec((SHARED_FF, D_MODEL), lambda i: (0, 0)),
        ],
        out_specs=[
            pl.BlockSpec((TILE, D_MODEL), lambda i: (i, 0)),
            pl.BlockSpec((8, TILE), lambda i: (0, i)),
            pl.BlockSpec((TILE, D_MODEL // 2), lambda i: (i, 0)),
            pl.BlockSpec((TILE, 16), lambda i: (i, 0)),
            pl.BlockSpec((TILE, 16), lambda i: (i, 0)),
            pl.BlockSpec((1, META_W), lambda i: (0, 0)),
        ],
        out_shape=[
            jax.ShapeDtypeStruct((T, D_MODEL), jnp.float32),
            jax.ShapeDtypeStruct((8, T), jnp.int32),
            jax.ShapeDtypeStruct((T, D_MODEL // 2), jnp.int32),
            jax.ShapeDtypeStruct((T, 16), jnp.float32),
            jax.ShapeDtypeStruct((T, 16), jnp.float32),
            jax.ShapeDtypeStruct((1, META_W), jnp.int32),
        ],
        scratch_shapes=[pltpu.VMEM((1, E), jnp.float32)],
    )(hidden, gate_wt, sgu, sdown)


def _dispatch_body(hid_hbm, s0_hbm, s1_hbm, xs_hbm,
                   hid_a, hid_b, s0a, s0b, s1a, s1b, sem_r, sem_w):
    wid = lax.axis_index("s") * 2 + lax.axis_index("c")
    base = wid * TOK_PER_W
    ra = pltpu.make_async_copy(hid_hbm.at[pl.ds(base, HALF_W)], hid_a, sem_r)
    ra.start()
    rb = pltpu.make_async_copy(
        hid_hbm.at[pl.ds(base + HALF_W, HALF_W)], hid_b, sem_r)
    rb.start()
    pltpu.sync_copy(s0_hbm.at[pl.ds(base, HALF_W)], s0a)
    pltpu.sync_copy(s0_hbm.at[pl.ds(base + HALF_W, HALF_W)], s0b)
    pltpu.sync_copy(s1_hbm.at[pl.ds(base, HALF_W)], s1a)
    pltpu.sync_copy(s1_hbm.at[pl.ds(base + HALF_W, HALF_W)], s1b)
    ra.wait()
    wa0 = pltpu.make_async_copy(hid_a, xs_hbm.at[s0a], sem_w)
    wa0.start()
    wa1 = pltpu.make_async_copy(hid_a, xs_hbm.at[s1a], sem_w)
    wa1.start()
    rb.wait()
    wb0 = pltpu.make_async_copy(hid_b, xs_hbm.at[s0b], sem_w)
    wb0.start()
    wb1 = pltpu.make_async_copy(hid_b, xs_hbm.at[s1b], sem_w)
    wb1.start()
    wa0.wait()
    wa1.wait()
    wb0.wait()
    wb1.wait()


def _dispatch(hidden, s0, s1):
    mesh = plsc.VectorSubcoreMesh(core_axis_name="c", subcore_axis_name="s",
                                  num_cores=2, num_subcores=16)
    f = pl.kernel(
        _dispatch_body,
        out_type=jax.ShapeDtypeStruct((N_BUF, D_MODEL // 2), jnp.int32),
        mesh=mesh,
        scratch_types=[
            pltpu.VMEM((HALF_W, D_MODEL // 2), jnp.int32),
            pltpu.VMEM((HALF_W, D_MODEL // 2), jnp.int32),
            pltpu.VMEM((HALF_W,), jnp.int32),
            pltpu.VMEM((HALF_W,), jnp.int32),
            pltpu.VMEM((HALF_W,), jnp.int32),
            pltpu.VMEM((HALF_W,), jnp.int32),
            pltpu.SemaphoreType.DMA,
            pltpu.SemaphoreType.DMA,
        ],
    )
    return f(hidden, s0, s1)


def _gmm_body(texp_ref, tblock_ref, x_ref, wgu_ref, wdown_ref, y_ref):
    i = pl.program_id(0)

    @pl.when(tblock_ref[i] != DUMP_BLOCK)
    def _():
        xb = _unpack(x_ref[...])
        h = jnp.dot(xb, wgu_ref[0].astype(jnp.bfloat16),
                    preferred_element_type=jnp.float32)
        g = h[:, :MOE_FF]
        u = h[:, MOE_FF:]
        act = (_silu(g) * u).astype(jnp.bfloat16)
        y = jnp.dot(act, wdown_ref[0].astype(jnp.bfloat16),
                    preferred_element_type=jnp.float32)
        y_ref[...] = _pack(y.astype(jnp.bfloat16))


def _gmm(texp, tblock, x_sorted, wgu, wdown):
    grid_spec = pltpu.PrefetchScalarGridSpec(
        num_scalar_prefetch=2,
        grid=(MAX_TILES,),
        in_specs=[
            pl.BlockSpec((TILE, D_MODEL // 2),
                         lambda i, texp, tblock: (tblock[i], 0)),
            pl.BlockSpec((1, D_MODEL, 2 * MOE_FF),
                         lambda i, texp, tblock: (texp[i], 0, 0)),
            pl.BlockSpec((1, MOE_FF, D_MODEL),
                         lambda i, texp, tblock: (texp[i], 0, 0)),
        ],
        out_specs=pl.BlockSpec((TILE, D_MODEL // 2),
                               lambda i, texp, tblock: (tblock[i], 0)),
    )
    return pl.pallas_call(
        _gmm_body,
        grid_spec=grid_spec,
        out_shape=jax.ShapeDtypeStruct((N_BUF, D_MODEL // 2), jnp.int32),
    )(texp, tblock, x_sorted, wgu, wdown)


def _combine_body(y_hbm, s0_hbm, s1_hbm, w0_hbm, w1_hbm, sh_hbm, out_hbm,
                  g0_a, g0_b, g1_a, g1_b, sh_a, sh_b,
                  s0_v, s1_v, w0_v, w1_v, sem_a, sem_b, sem_o):
    wid = lax.axis_index("s") * 2 + lax.axis_index("c")
    base = wid * TOK_PER_W
    pltpu.sync_copy(s0_hbm.at[pl.ds(base, TOK_PER_W)], s0_v)
    pltpu.sync_copy(s1_hbm.at[pl.ds(base, TOK_PER_W)], s1_v)
    pltpu.sync_copy(w0_hbm.at[pl.ds(base, TOK_PER_W)], w0_v)
    pltpu.sync_copy(w1_hbm.at[pl.ds(base, TOK_PER_W)], w1_v)

    bufs = ((g0_a, g1_a, sh_a, sem_a), (g0_b, g1_b, sh_b, sem_b))
    out_pend = [None, None]

    def start(chunk):
        g0, g1, sh, sem = bufs[chunk % 2]
        if out_pend[chunk % 2] is not None:
            out_pend[chunk % 2].wait()
            out_pend[chunk % 2] = None
        idx0 = s0_v[pl.ds(chunk * CHUNK, CHUNK)]
        idx1 = s1_v[pl.ds(chunk * CHUNK, CHUNK)]
        cs = (pltpu.make_async_copy(y_hbm.at[idx0], g0, sem),
              pltpu.make_async_copy(y_hbm.at[idx1], g1, sem),
              pltpu.make_async_copy(
                  sh_hbm.at[pl.ds(base + chunk * CHUNK, CHUNK)], sh, sem))
        for c in cs:
            c.start()
        return cs

    pend = start(0)
    for chunk in range(N_CHUNK):
        g0, g1, sh, sem = bufs[chunk % 2]
        for c in pend:
            c.wait()
        if chunk + 1 < N_CHUNK:
            pend = start(chunk + 1)

        def tok(j, _):
            w0s = w0_v[j + chunk * CHUNK, :]
            w1s = w1_v[j + chunk * CHUNK, :]
            for c in range(D_MODEL // 32):
                sl = pl.ds(c * 16, 16)
                slh = pl.ds(D_MODEL // 2 + c * 16, 16)
                v0 = g0[j, sl]
                v1 = g1[j, sl]
                lo0 = jax.lax.bitcast_convert_type(v0 << 16, jnp.float32)
                hi0 = jax.lax.bitcast_convert_type(v0 & jnp.int32(-65536),
                                                   jnp.float32)
                lo1 = jax.lax.bitcast_convert_type(v1 << 16, jnp.float32)
                hi1 = jax.lax.bitcast_convert_type(v1 & jnp.int32(-65536),
                                                   jnp.float32)
                sh[j, sl] = sh[j, sl] + w0s * lo0 + w1s * lo1
                sh[j, slh] = sh[j, slh] + w0s * hi0 + w1s * hi1
            return 0

        lax.fori_loop(0, CHUNK, tok, 0)
        oc = pltpu.make_async_copy(
            sh, out_hbm.at[pl.ds(base + chunk * CHUNK, CHUNK)], sem_o)
        oc.start()
        out_pend[chunk % 2] = oc
    for oc in out_pend:
        if oc is not None:
            oc.wait()


def _combine(y, s0, s1, w0e, w1e, shared_out):
    mesh = plsc.VectorSubcoreMesh(core_axis_name="c", subcore_axis_name="s",
                                  num_cores=2, num_subcores=16)
    f = pl.kernel(
        _combine_body,
        out_type=jax.ShapeDtypeStruct((T, D_MODEL), jnp.float32),
        mesh=mesh,
        scratch_types=[
            pltpu.VMEM((CHUNK, D_MODEL // 2), jnp.int32),
            pltpu.VMEM((CHUNK, D_MODEL // 2), jnp.int32),
            pltpu.VMEM((CHUNK, D_MODEL // 2), jnp.int32),
            pltpu.VMEM((CHUNK, D_MODEL // 2), jnp.int32),
            pltpu.VMEM((CHUNK, D_MODEL), jnp.float32),
            pltpu.VMEM((CHUNK, D_MODEL), jnp.float32),
            pltpu.VMEM((TOK_PER_W,), jnp.int32),
            pltpu.VMEM((TOK_PER_W,), jnp.int32),
            pltpu.VMEM((TOK_PER_W, 16), jnp.float32),
            pltpu.VMEM((TOK_PER_W, 16), jnp.float32),
            pltpu.SemaphoreType.DMA,
            pltpu.SemaphoreType.DMA,
            pltpu.SemaphoreType.DMA,
        ],
    )
    return f(y, s0, s1, w0e, w1e, shared_out)


@jax.jit
def kernel(hidden_states, gate_w, w_gate_up, w_down, shared_gate_up,
           shared_down):
    gate_wt = gate_w.T

    shared_out, slots, xpk, w0e, w1e, meta = _head(hidden_states, gate_wt,
                                                   shared_gate_up,
                                                   shared_down)
    s0 = slots[0]
    s1 = slots[1]
    texp = meta[0, :MAX_TILES_PAD]
    tblock = meta[0, MAX_TILES_PAD:2 * MAX_TILES_PAD]

    x_sorted = _dispatch(xpk, s0, s1)
    y = _gmm(texp, tblock, x_sorted, w_gate_up, w_down)
    return _combine(y, s0, s1, w0e, w1e, shared_out)
